# baseline (speedup 1.0000x reference)
import functools

import jax
import jax.numpy as jnp
from jax import lax
from jax.experimental import pallas as pl
from jax.experimental.pallas import tpu as pltpu
from jax.experimental.pallas import tpu_sc as plsc

VOCAB = 100000
DIM = 64
B = 4096
KN = 20
K = 2 * KN + 1
NC = 2
NS = 16
NW = NC * NS
BW = B // NW
NG = BW // 16


def _sc_body(idx_all, pair_v, u_table, v_table, out,
             idxbuf, vidx, vrows, urows, dotbuf, semv, semg):
    wid = lax.axis_index("s") * NC + lax.axis_index("c")
    base = wid * BW

    pltpu.sync_copy(idx_all.at[:, pl.ds(base, BW)], idxbuf)
    pltpu.sync_copy(pair_v.at[pl.ds(base, BW)], vidx)
    pltpu.async_copy(v_table.at[vidx], vrows, semv).wait()

    def kbody(k, carry):
        pltpu.async_copy(u_table.at[idxbuf.at[k]], urows, semg).wait()

        def gbody(g, c):
            riota = lax.broadcasted_iota(jnp.int32, (16,), 0) + g * 16
            acc = jnp.zeros((16,), jnp.float32)
            for d in range(DIM):
                dv = jnp.full((16,), d, jnp.int32)
                acc = acc + (plsc.load_gather(urows, [riota, dv]) *
                             plsc.load_gather(vrows, [riota, dv]))
            dotbuf[k, pl.ds(g * 16, 16)] = acc
            return c

        return lax.fori_loop(0, NG, gbody, carry)

    lax.fori_loop(0, K, kbody, 0)
    pltpu.sync_copy(dotbuf, out.at[:, pl.ds(base, BW)])


@jax.jit
def _sc_dots(idx_all, pair_v, u_table, v_table):
    mesh = plsc.VectorSubcoreMesh(core_axis_name="c", subcore_axis_name="s")
    return pl.kernel(
        _sc_body,
        out_type=jax.ShapeDtypeStruct((K, B), jnp.float32),
        mesh=mesh,
        compiler_params=pltpu.CompilerParams(
            needs_layout_passes=False, use_tc_tiling_on_sc=False),
        scratch_types=[
            pltpu.VMEM((K, BW), jnp.int32),
            pltpu.VMEM((BW,), jnp.int32),
            pltpu.VMEM((BW, DIM), jnp.float32),
            pltpu.VMEM((BW, DIM), jnp.float32),
            pltpu.VMEM((K, BW), jnp.float32),
            pltpu.SemaphoreType.DMA,
            pltpu.SemaphoreType.DMA,
        ],
    )(idx_all, pair_v, u_table, v_table)


def _fin_body(dots_ref, mask_ref, out_ref):
    x = dots_ref[...]
    m = mask_ref[...]
    row = lax.broadcasted_iota(jnp.int32, (K, B), 0)
    mm = x * m
    s = jnp.where(row >= 1 + KN, -mm, mm)
    t = jnp.minimum(s, 0.0) - jnp.log1p(jnp.exp(-jnp.abs(s)))
    coef = jnp.where(row == 0, jnp.float32(1 + KN),
                     jnp.where(row >= 1 + KN, jnp.float32(1.0),
                               jnp.float32(-1.0)))
    out_ref[0, 0] = -jnp.sum(coef * t)


def _finish(dots, mask_all):
    return pl.pallas_call(
        _fin_body,
        out_shape=jax.ShapeDtypeStruct((1, 1), jnp.float32),
        in_specs=[
            pl.BlockSpec(memory_space=pltpu.VMEM),
            pl.BlockSpec(memory_space=pltpu.VMEM),
        ],
        out_specs=pl.BlockSpec(memory_space=pltpu.SMEM),
    )(dots, mask_all)


def kernel(pair_u, pair_v, pos_u, mask_pos_u, neg_u, mask_neg_u,
           u_table, v_table):
    pair_u = pair_u.astype(jnp.int32)
    pair_v = pair_v.astype(jnp.int32)
    idx_all = jnp.concatenate(
        [pair_u[None, :], pos_u.astype(jnp.int32).T,
         neg_u.astype(jnp.int32).T], axis=0)
    mask_all = jnp.concatenate(
        [jnp.ones((1, B), jnp.float32), mask_pos_u.T, mask_neg_u.T], axis=0)

    dots = _sc_dots(idx_all, pair_v, u_table, v_table)
    return _finish(dots, mask_all)[0, 0]


# speedup vs baseline: 1.0617x; 1.0617x over previous
import jax
import jax.numpy as jnp
from jax import lax
from jax.experimental import pallas as pl
from jax.experimental.pallas import tpu as pltpu
from jax.experimental.pallas import tpu_sc as plsc

VOCAB = 100000
DIM = 64
B = 4096
KN = 20
K = 2 * KN + 1
NC = 2
NS = 16
NW = NC * NS
BW = B // NW
NG = BW // 16
NBUF = 4


def _iota16():
    return lax.broadcasted_iota(jnp.int32, (16,), 0)


def _sc_body(pair_u, pair_v, pos_u, neg_u, mask_pos, mask_neg,
             u_table, v_table, out,
             idxbuf, maskbuf, pvbuf, ps, ns, mp, mn, vrows, mdotbuf,
             ubufs, semv, sems):
    wid = lax.axis_index("s") * NC + lax.axis_index("c")
    base = wid * BW
    bsl = pl.ds(base, BW)

    pltpu.sync_copy(pair_u.at[bsl], idxbuf.at[0])
    pltpu.sync_copy(pair_v.at[bsl], pvbuf)
    cpv = pltpu.async_copy(v_table.at[pvbuf], vrows, semv)
    pltpu.sync_copy(pos_u.at[bsl], ps)
    pltpu.sync_copy(neg_u.at[bsl], ns)
    pltpu.sync_copy(mask_pos.at[bsl], mp)
    pltpu.sync_copy(mask_neg.at[bsl], mn)

    for g in range(NG):
        maskbuf[0, pl.ds(g * 16, 16)] = jnp.full((16,), 1.0, jnp.float32)

    def tbody(k, c):
        kv = jnp.full((16,), 0, jnp.int32) + k
        for g in range(NG):
            biota = _iota16() + g * 16
            gsl = pl.ds(g * 16, 16)
            idxbuf[1 + k, gsl] = plsc.load_gather(ps, [biota, kv])
            idxbuf[1 + KN + k, gsl] = plsc.load_gather(ns, [biota, kv])
            maskbuf[1 + k, gsl] = plsc.load_gather(mp, [biota, kv])
            maskbuf[1 + KN + k, gsl] = plsc.load_gather(mn, [biota, kv])
        return c

    lax.fori_loop(0, KN, tbody, 0)
    cpv.wait()

    def start_gather(k, j):
        pltpu.async_copy(u_table.at[idxbuf.at[k]], ubufs[j], sems[j])

    def wait_gather(k, j):
        pltpu.make_async_copy(u_table.at[idxbuf.at[k]], ubufs[j],
                              sems[j]).wait()

    def compute_k(rows, k):
        def gbody(g, c):
            riota = _iota16() + g * 16
            acc = jnp.zeros((16,), jnp.float32)
            for d in range(DIM):
                dv = jnp.full((16,), d, jnp.int32)
                acc = acc + (plsc.load_gather(rows, [riota, dv]) *
                             plsc.load_gather(vrows, [riota, dv]))
            gsl = pl.ds(g * 16, 16)
            mdotbuf[k, gsl] = acc * maskbuf[k, gsl]
            return c
        lax.fori_loop(0, NG, gbody, 0)

    for j in range(NBUF):
        start_gather(j, j)

    def kbody(s, c):
        for j in range(NBUF):
            k = NBUF * s + j
            wait_gather(k, j)
            compute_k(ubufs[j], k)
            knext = k + NBUF

            @pl.when(knext <= K - 1)
            def _():
                start_gather(knext, j)
        return c

    lax.fori_loop(0, (K - 1) // NBUF, kbody, 0)
    wait_gather(K - 1, 0)
    compute_k(ubufs[0], K - 1)

    pltpu.sync_copy(mdotbuf, out.at[:, bsl])


@jax.jit
def _sc_dots(pair_u, pair_v, pos_u, neg_u, mask_pos, mask_neg,
             u_table, v_table):
    mesh = plsc.VectorSubcoreMesh(core_axis_name="c", subcore_axis_name="s")
    return pl.kernel(
        _sc_body,
        out_type=jax.ShapeDtypeStruct((K, B), jnp.float32),
        mesh=mesh,
        compiler_params=pltpu.CompilerParams(
            needs_layout_passes=False, use_tc_tiling_on_sc=False),
        scratch_types=[
            pltpu.VMEM((K, BW), jnp.int32),
            pltpu.VMEM((K, BW), jnp.float32),
            pltpu.VMEM((BW,), jnp.int32),
            pltpu.VMEM((BW, KN), jnp.int32),
            pltpu.VMEM((BW, KN), jnp.int32),
            pltpu.VMEM((BW, KN), jnp.float32),
            pltpu.VMEM((BW, KN), jnp.float32),
            pltpu.VMEM((BW, DIM), jnp.float32),
            pltpu.VMEM((K, BW), jnp.float32),
            [pltpu.VMEM((BW, DIM), jnp.float32) for _ in range(NBUF)],
            pltpu.SemaphoreType.DMA,
            [pltpu.SemaphoreType.DMA for _ in range(NBUF)],
        ],
    )(pair_u, pair_v, pos_u, neg_u, mask_pos, mask_neg, u_table, v_table)


def _fin_body(dots_ref, out_ref):
    x = dots_ref[...]
    row = lax.broadcasted_iota(jnp.int32, (K, B), 0)
    s = jnp.where(row >= 1 + KN, -x, x)
    t = jnp.minimum(s, 0.0) - jnp.log1p(jnp.exp(-jnp.abs(s)))
    coef = jnp.where(row == 0, jnp.float32(1 + KN),
                     jnp.where(row >= 1 + KN, jnp.float32(1.0),
                               jnp.float32(-1.0)))
    out_ref[0, 0] = -jnp.sum(coef * t)


def _finish(dots):
    return pl.pallas_call(
        _fin_body,
        out_shape=jax.ShapeDtypeStruct((1, 1), jnp.float32),
        in_specs=[pl.BlockSpec(memory_space=pltpu.VMEM)],
        out_specs=pl.BlockSpec(memory_space=pltpu.SMEM),
    )(dots)


def kernel(pair_u, pair_v, pos_u, mask_pos_u, neg_u, mask_neg_u,
           u_table, v_table):
    dots = _sc_dots(pair_u.astype(jnp.int32), pair_v.astype(jnp.int32),
                    pos_u.astype(jnp.int32), neg_u.astype(jnp.int32),
                    mask_pos_u, mask_neg_u, u_table, v_table)
    return _finish(dots)[0, 0]


# speedup vs baseline: 1.4824x; 1.3962x over previous
import jax
import jax.numpy as jnp
from jax import lax
from jax.experimental import pallas as pl
from jax.experimental.pallas import tpu as pltpu
from jax.experimental.pallas import tpu_sc as plsc

VOCAB = 100000
DIM = 64
B = 4096
KN = 20
K = 2 * KN + 1
NC = 2
NS = 16
NW = NC * NS
BW = B // NW
NG = BW // 16
GK = 4
NGRP = (K - 1) // GK


def _iota16():
    return lax.broadcasted_iota(jnp.int32, (16,), 0)


def _sc_body(pair_u, pair_v, pos_u, neg_u, mask_pos, mask_neg,
             u_table, v_table, out,
             idxbuf, maskbuf, pvbuf, ps, ns, mp, mn, vrows, vt, mdotbuf,
             abufs, bbufs, semv, semp, asems, bsems):
    wid = lax.axis_index("s") * NC + lax.axis_index("c")
    base = wid * BW
    bsl = pl.ds(base, BW)

    pltpu.sync_copy(pair_u.at[bsl], idxbuf.at[0])
    pltpu.sync_copy(pair_v.at[bsl], pvbuf)
    cpv = pltpu.async_copy(v_table.at[pvbuf], vrows, semv)
    cpp = pltpu.async_copy(u_table.at[idxbuf.at[0]], abufs[0], semp)
    pltpu.sync_copy(pos_u.at[bsl], ps)
    pltpu.sync_copy(neg_u.at[bsl], ns)
    pltpu.sync_copy(mask_pos.at[bsl], mp)
    pltpu.sync_copy(mask_neg.at[bsl], mn)

    for g in range(NG):
        maskbuf[0, pl.ds(g * 16, 16)] = jnp.full((16,), 1.0, jnp.float32)

    def tbody(k, c):
        kv = jnp.full((16,), 0, jnp.int32) + k
        for g in range(NG):
            biota = _iota16() + g * 16
            gsl = pl.ds(g * 16, 16)
            idxbuf[1 + k, gsl] = plsc.load_gather(ps, [biota, kv])
            idxbuf[1 + KN + k, gsl] = plsc.load_gather(ns, [biota, kv])
            maskbuf[1 + k, gsl] = plsc.load_gather(mp, [biota, kv])
            maskbuf[1 + KN + k, gsl] = plsc.load_gather(mn, [biota, kv])
        return c

    lax.fori_loop(0, KN, tbody, 0)

    def start_group(kbase, bufs, sems):
        for j in range(GK):
            pltpu.async_copy(u_table.at[idxbuf.at[kbase + j]], bufs[j],
                             sems[j])

    def wait_group(bufs, sems):
        for j in range(GK):
            pltpu.make_async_copy(u_table.at[idxbuf.at[0]], bufs[j],
                                  sems[j]).wait()

    start_group(1 + GK, bbufs, bsems)

    cpv.wait()

    def vtbody(d, c):
        dv = jnp.full((16,), 0, jnp.int32) + d
        for g in range(NG):
            riota = _iota16() + g * 16
            vt[d, pl.ds(g * 16, 16)] = plsc.load_gather(vrows, [riota, dv])
        return c

    lax.fori_loop(0, DIM, vtbody, 0)

    cpp.wait()

    def pbody(g, c):
        riota = _iota16() + g * 16
        gsl = pl.ds(g * 16, 16)
        acc0 = jnp.zeros((16,), jnp.float32)
        acc1 = jnp.zeros((16,), jnp.float32)
        for d in range(0, DIM, 2):
            acc0 = acc0 + (plsc.load_gather(abufs[0], [riota, jnp.full((16,), d, jnp.int32)]) *
                           vt[d, gsl])
            acc1 = acc1 + (plsc.load_gather(abufs[0], [riota, jnp.full((16,), d + 1, jnp.int32)]) *
                           vt[d + 1, gsl])
        mdotbuf[0, gsl] = acc0 + acc1
        return c

    lax.fori_loop(0, NG, pbody, 0)
    start_group(1, abufs, asems)

    def compute_group(kbase, bufs):
        def gbody(g, c):
            riota = _iota16() + g * 16
            gsl = pl.ds(g * 16, 16)
            accs = [jnp.zeros((16,), jnp.float32) for _ in range(GK)]
            for d in range(DIM):
                dv = jnp.full((16,), d, jnp.int32)
                vc = vt[d, gsl]
                for j in range(GK):
                    accs[j] = accs[j] + plsc.load_gather(bufs[j], [riota, dv]) * vc
            for j in range(GK):
                mdotbuf[kbase + j, gsl] = accs[j] * maskbuf[kbase + j, gsl]
            return c
        lax.fori_loop(0, NG, gbody, 0)

    def sbody(s2, c):
        ka = 8 * s2 + 1
        wait_group(abufs, asems)
        compute_group(ka, abufs)

        @pl.when(ka + 8 <= K - GK)
        def _():
            start_group(ka + 8, abufs, asems)

        kb = ka + GK
        wait_group(bbufs, bsems)
        compute_group(kb, bbufs)

        @pl.when(kb + 8 <= K - GK)
        def _():
            start_group(kb + 8, bbufs, bsems)
        return c

    lax.fori_loop(0, NGRP // 2, sbody, 0)

    pltpu.sync_copy(mdotbuf, out.at[:, bsl])


@jax.jit
def _sc_dots(pair_u, pair_v, pos_u, neg_u, mask_pos, mask_neg,
             u_table, v_table):
    mesh = plsc.VectorSubcoreMesh(core_axis_name="c", subcore_axis_name="s")
    return pl.kernel(
        _sc_body,
        out_type=jax.ShapeDtypeStruct((K, B), jnp.float32),
        mesh=mesh,
        compiler_params=pltpu.CompilerParams(
            needs_layout_passes=False, use_tc_tiling_on_sc=False),
        scratch_types=[
            pltpu.VMEM((K, BW), jnp.int32),
            pltpu.VMEM((K, BW), jnp.float32),
            pltpu.VMEM((BW,), jnp.int32),
            pltpu.VMEM((BW, KN), jnp.int32),
            pltpu.VMEM((BW, KN), jnp.int32),
            pltpu.VMEM((BW, KN), jnp.float32),
            pltpu.VMEM((BW, KN), jnp.float32),
            pltpu.VMEM((BW, DIM), jnp.float32),
            pltpu.VMEM((DIM, BW), jnp.float32),
            pltpu.VMEM((K, BW), jnp.float32),
            [pltpu.VMEM((BW, DIM), jnp.float32) for _ in range(GK)],
            [pltpu.VMEM((BW, DIM), jnp.float32) for _ in range(GK)],
            pltpu.SemaphoreType.DMA,
            pltpu.SemaphoreType.DMA,
            [pltpu.SemaphoreType.DMA for _ in range(GK)],
            [pltpu.SemaphoreType.DMA for _ in range(GK)],
        ],
    )(pair_u, pair_v, pos_u, neg_u, mask_pos, mask_neg, u_table, v_table)


def _fin_body(dots_ref, out_ref):
    x = dots_ref[...]
    row = lax.broadcasted_iota(jnp.int32, (K, B), 0)
    s = jnp.where(row >= 1 + KN, -x, x)
    t = jnp.minimum(s, 0.0) - jnp.log1p(jnp.exp(-jnp.abs(s)))
    coef = jnp.where(row == 0, jnp.float32(1 + KN),
                     jnp.where(row >= 1 + KN, jnp.float32(1.0),
                               jnp.float32(-1.0)))
    out_ref[0, 0] = -jnp.sum(coef * t)


def _finish(dots):
    return pl.pallas_call(
        _fin_body,
        out_shape=jax.ShapeDtypeStruct((1, 1), jnp.float32),
        in_specs=[pl.BlockSpec(memory_space=pltpu.VMEM)],
        out_specs=pl.BlockSpec(memory_space=pltpu.SMEM),
    )(dots)


def kernel(pair_u, pair_v, pos_u, mask_pos_u, neg_u, mask_neg_u,
           u_table, v_table):
    dots = _sc_dots(pair_u.astype(jnp.int32), pair_v.astype(jnp.int32),
                    pos_u.astype(jnp.int32), neg_u.astype(jnp.int32),
                    mask_pos_u, mask_neg_u, u_table, v_table)
    return _finish(dots)[0, 0]


# speedup vs baseline: 2.7403x; 1.8486x over previous
import jax
import jax.numpy as jnp
from jax import lax
from jax.experimental import pallas as pl
from jax.experimental.pallas import tpu as pltpu
from jax.experimental.pallas import tpu_sc as plsc

VOCAB = 100000
DIM = 64
B = 4096
KN = 20
K = 2 * KN + 1
NC = 2
NS = 16
NW = NC * NS
BW = B // NW
NG = BW // 16
GK = 4
NGRP = (K - 1) // GK


def _iota16():
    return lax.broadcasted_iota(jnp.int32, (16,), 0)


def _sc_body(pair_u, pair_v, pos_u, neg_u, mask_pos, mask_neg,
             u_table, v_table, out,
             idxbuf, maskbuf, pvbuf, ps, ns, mp, mn, vrows, mdotbuf,
             abufs, bbufs, semv, semp, asems, bsems):
    wid = lax.axis_index("s") * NC + lax.axis_index("c")
    base = wid * BW
    bsl = pl.ds(base, BW)

    pltpu.sync_copy(pair_u.at[bsl], idxbuf.at[0])
    pltpu.sync_copy(pair_v.at[bsl], pvbuf)
    cpv = pltpu.async_copy(v_table.at[pvbuf], vrows, semv)
    cpp = pltpu.async_copy(u_table.at[idxbuf.at[0]], abufs[0], semp)
    pltpu.sync_copy(pos_u.at[bsl], ps)
    pltpu.sync_copy(neg_u.at[bsl], ns)
    pltpu.sync_copy(mask_pos.at[bsl], mp)
    pltpu.sync_copy(mask_neg.at[bsl], mn)

    for g in range(NG):
        maskbuf[0, pl.ds(g * 16, 16)] = jnp.full((16,), 1.0, jnp.float32)

    def tbody(k, c):
        kv = jnp.full((16,), 0, jnp.int32) + k
        for g in range(NG):
            biota = _iota16() + g * 16
            gsl = pl.ds(g * 16, 16)
            idxbuf[1 + k, gsl] = plsc.load_gather(ps, [biota, kv])
            idxbuf[1 + KN + k, gsl] = plsc.load_gather(ns, [biota, kv])
            maskbuf[1 + k, gsl] = plsc.load_gather(mp, [biota, kv])
            maskbuf[1 + KN + k, gsl] = plsc.load_gather(mn, [biota, kv])
        return c

    lax.fori_loop(0, KN, tbody, 0)

    def start_group(kbase, bufs, sems):
        for j in range(GK):
            pltpu.async_copy(u_table.at[idxbuf.at[kbase + j]], bufs[j],
                             sems[j])

    def wait_group(bufs, sems):
        for j in range(GK):
            pltpu.make_async_copy(u_table.at[idxbuf.at[0]], bufs[j],
                                  sems[j]).wait()

    start_group(1 + GK, bbufs, bsems)

    cpv.wait()
    cpp.wait()

    def pbody(g, c):
        riota = _iota16() + g * 16
        gsl = pl.ds(g * 16, 16)
        acc0 = jnp.zeros((16,), jnp.float32)
        acc1 = jnp.zeros((16,), jnp.float32)
        col = _iota16()
        for d in range(0, DIM, 2):
            acc0 = acc0 + (plsc.load_gather(abufs[0], [riota, col]) *
                           plsc.load_gather(vrows, [riota, col]))
            col1 = (col + 1) & (DIM - 1)
            acc1 = acc1 + (plsc.load_gather(abufs[0], [riota, col1]) *
                           plsc.load_gather(vrows, [riota, col1]))
            col = (col1 + 1) & (DIM - 1)
        mdotbuf[0, gsl] = acc0 + acc1
        return c

    lax.fori_loop(0, NG, pbody, 0)
    start_group(1, abufs, asems)

    def compute_group(kbase, bufs):
        def gbody(g, c):
            riota = _iota16() + g * 16
            gsl = pl.ds(g * 16, 16)
            accs = [jnp.zeros((16,), jnp.float32) for _ in range(GK)]
            col = _iota16()
            for d in range(DIM):
                vc = plsc.load_gather(vrows, [riota, col])
                for j in range(GK):
                    accs[j] = accs[j] + plsc.load_gather(bufs[j], [riota, col]) * vc
                col = (col + 1) & (DIM - 1)
            for j in range(GK):
                mdotbuf[kbase + j, gsl] = accs[j] * maskbuf[kbase + j, gsl]
            return c
        lax.fori_loop(0, NG, gbody, 0)

    def sbody(s2, c):
        ka = 8 * s2 + 1
        wait_group(abufs, asems)
        compute_group(ka, abufs)

        @pl.when(ka + 8 <= K - GK)
        def _():
            start_group(ka + 8, abufs, asems)

        kb = ka + GK
        wait_group(bbufs, bsems)
        compute_group(kb, bbufs)

        @pl.when(kb + 8 <= K - GK)
        def _():
            start_group(kb + 8, bbufs, bsems)
        return c

    lax.fori_loop(0, NGRP // 2, sbody, 0)

    pltpu.sync_copy(mdotbuf, out.at[:, bsl])


@jax.jit
def _sc_dots(pair_u, pair_v, pos_u, neg_u, mask_pos, mask_neg,
             u_table, v_table):
    mesh = plsc.VectorSubcoreMesh(core_axis_name="c", subcore_axis_name="s")
    return pl.kernel(
        _sc_body,
        out_type=jax.ShapeDtypeStruct((K, B), jnp.float32),
        mesh=mesh,
        compiler_params=pltpu.CompilerParams(
            needs_layout_passes=False, use_tc_tiling_on_sc=False),
        scratch_types=[
            pltpu.VMEM((K, BW), jnp.int32),
            pltpu.VMEM((K, BW), jnp.float32),
            pltpu.VMEM((BW,), jnp.int32),
            pltpu.VMEM((BW, KN), jnp.int32),
            pltpu.VMEM((BW, KN), jnp.int32),
            pltpu.VMEM((BW, KN), jnp.float32),
            pltpu.VMEM((BW, KN), jnp.float32),
            pltpu.VMEM((BW, DIM), jnp.float32),
            pltpu.VMEM((K, BW), jnp.float32),
            [pltpu.VMEM((BW, DIM), jnp.float32) for _ in range(GK)],
            [pltpu.VMEM((BW, DIM), jnp.float32) for _ in range(GK)],
            pltpu.SemaphoreType.DMA,
            pltpu.SemaphoreType.DMA,
            [pltpu.SemaphoreType.DMA for _ in range(GK)],
            [pltpu.SemaphoreType.DMA for _ in range(GK)],
        ],
    )(pair_u, pair_v, pos_u, neg_u, mask_pos, mask_neg, u_table, v_table)


def _fin_body(dots_ref, out_ref):
    x = dots_ref[...]
    row = lax.broadcasted_iota(jnp.int32, (K, B), 0)
    s = jnp.where(row >= 1 + KN, -x, x)
    t = jnp.minimum(s, 0.0) - jnp.log1p(jnp.exp(-jnp.abs(s)))
    coef = jnp.where(row == 0, jnp.float32(1 + KN),
                     jnp.where(row >= 1 + KN, jnp.float32(1.0),
                               jnp.float32(-1.0)))
    out_ref[0, 0] = -jnp.sum(coef * t)


def _finish(dots):
    return pl.pallas_call(
        _fin_body,
        out_shape=jax.ShapeDtypeStruct((1, 1), jnp.float32),
        in_specs=[pl.BlockSpec(memory_space=pltpu.VMEM)],
        out_specs=pl.BlockSpec(memory_space=pltpu.SMEM),
    )(dots)


def kernel(pair_u, pair_v, pos_u, mask_pos_u, neg_u, mask_neg_u,
           u_table, v_table):
    dots = _sc_dots(pair_u.astype(jnp.int32), pair_v.astype(jnp.int32),
                    pos_u.astype(jnp.int32), neg_u.astype(jnp.int32),
                    mask_pos_u, mask_neg_u, u_table, v_table)
    return _finish(dots)[0, 0]


# speedup vs baseline: 2.7666x; 1.0096x over previous
import jax
import jax.numpy as jnp
from jax import lax
from jax.experimental import pallas as pl
from jax.experimental.pallas import tpu as pltpu
from jax.experimental.pallas import tpu_sc as plsc

VOCAB = 100000
DIM = 64
B = 4096
KN = 20
K = 2 * KN + 1
NC = 2
NS = 16
NW = NC * NS
BW = B // NW
NG = BW // 16
GK = 5
NGRP = (K - 1) // GK


def _iota16():
    return lax.broadcasted_iota(jnp.int32, (16,), 0)


def _sc_body(pair_u, pair_v, pos_u, neg_u, mask_pos, mask_neg,
             u_table, v_table, out,
             idxbuf, maskbuf, pvbuf, ps, ns, mp, mn, vrows, mdotbuf,
             abuf, bbuf, semv, semp, asem, bsem):
    wid = lax.axis_index("s") * NC + lax.axis_index("c")
    base = wid * BW
    bsl = pl.ds(base, BW)

    pltpu.sync_copy(pair_u.at[bsl], idxbuf.at[pl.ds(0, BW)])
    pltpu.sync_copy(pair_v.at[bsl], pvbuf)
    cpv = pltpu.async_copy(v_table.at[pvbuf], vrows, semv)
    cpp = pltpu.async_copy(u_table.at[idxbuf.at[pl.ds(0, BW)]],
                           abuf.at[pl.ds(0, BW)], semp)
    pltpu.sync_copy(pos_u.at[bsl], ps)
    pltpu.sync_copy(neg_u.at[bsl], ns)
    pltpu.sync_copy(mask_pos.at[bsl], mp)
    pltpu.sync_copy(mask_neg.at[bsl], mn)

    for g in range(NG):
        maskbuf[0, pl.ds(g * 16, 16)] = jnp.full((16,), 1.0, jnp.float32)

    def tbody(k, c):
        kv = jnp.full((16,), 0, jnp.int32) + k
        for g in range(NG):
            biota = _iota16() + g * 16
            gsl = pl.ds(g * 16, 16)
            idxbuf[pl.ds((1 + k) * BW + g * 16, 16)] = plsc.load_gather(
                ps, [biota, kv])
            idxbuf[pl.ds((1 + KN + k) * BW + g * 16, 16)] = plsc.load_gather(
                ns, [biota, kv])
            maskbuf[1 + k, gsl] = plsc.load_gather(mp, [biota, kv])
            maskbuf[1 + KN + k, gsl] = plsc.load_gather(mn, [biota, kv])
        return c

    lax.fori_loop(0, KN, tbody, 0)

    def start_group(kbase, buf, sem):
        pltpu.async_copy(u_table.at[idxbuf.at[pl.ds(kbase * BW, GK * BW)]],
                         buf, sem)

    def wait_group(buf, sem):
        pltpu.make_async_copy(u_table.at[idxbuf.at[pl.ds(BW, GK * BW)]], buf,
                              sem).wait()

    start_group(1 + GK, bbuf, bsem)

    cpv.wait()
    cpp.wait()

    def pbody(g, c):
        riota = _iota16() + g * 16
        gsl = pl.ds(g * 16, 16)
        acc0 = jnp.zeros((16,), jnp.float32)
        acc1 = jnp.zeros((16,), jnp.float32)
        col = _iota16()
        for d in range(0, DIM, 2):
            acc0 = acc0 + (plsc.load_gather(abuf, [riota, col]) *
                           plsc.load_gather(vrows, [riota, col]))
            col1 = (col + 1) & (DIM - 1)
            acc1 = acc1 + (plsc.load_gather(abuf, [riota, col1]) *
                           plsc.load_gather(vrows, [riota, col1]))
            col = (col1 + 1) & (DIM - 1)
        mdotbuf[0, gsl] = acc0 + acc1
        return c

    lax.fori_loop(0, NG, pbody, 0)
    start_group(1, abuf, asem)

    def compute_group(kbase, buf):
        def gbody(g, c):
            riota = _iota16() + g * 16
            gsl = pl.ds(g * 16, 16)
            accs = [jnp.zeros((16,), jnp.float32) for _ in range(GK)]
            riotas = [riota + j * BW for j in range(GK)]
            col = _iota16()
            for d in range(DIM):
                vc = plsc.load_gather(vrows, [riota, col])
                for j in range(GK):
                    accs[j] = accs[j] + plsc.load_gather(buf, [riotas[j], col]) * vc
                col = (col + 1) & (DIM - 1)
            for j in range(GK):
                mdotbuf[kbase + j, gsl] = accs[j] * maskbuf[kbase + j, gsl]
            return c
        lax.fori_loop(0, NG, gbody, 0)

    def sbody(s2, c):
        ka = 2 * GK * s2 + 1
        wait_group(abuf, asem)
        compute_group(ka, abuf)

        @pl.when(ka + 2 * GK <= K - GK)
        def _():
            start_group(ka + 2 * GK, abuf, asem)

        kb = ka + GK
        wait_group(bbuf, bsem)
        compute_group(kb, bbuf)

        @pl.when(kb + 2 * GK <= K - GK)
        def _():
            start_group(kb + 2 * GK, bbuf, bsem)
        return c

    lax.fori_loop(0, NGRP // 2, sbody, 0)

    pltpu.sync_copy(mdotbuf, out.at[:, bsl])


@jax.jit
def _sc_dots(pair_u, pair_v, pos_u, neg_u, mask_pos, mask_neg,
             u_table, v_table):
    mesh = plsc.VectorSubcoreMesh(core_axis_name="c", subcore_axis_name="s")
    return pl.kernel(
        _sc_body,
        out_type=jax.ShapeDtypeStruct((K, B), jnp.float32),
        mesh=mesh,
        compiler_params=pltpu.CompilerParams(
            needs_layout_passes=False, use_tc_tiling_on_sc=False),
        scratch_types=[
            pltpu.VMEM((K * BW,), jnp.int32),
            pltpu.VMEM((K, BW), jnp.float32),
            pltpu.VMEM((BW,), jnp.int32),
            pltpu.VMEM((BW, KN), jnp.int32),
            pltpu.VMEM((BW, KN), jnp.int32),
            pltpu.VMEM((BW, KN), jnp.float32),
            pltpu.VMEM((BW, KN), jnp.float32),
            pltpu.VMEM((BW, DIM), jnp.float32),
            pltpu.VMEM((K, BW), jnp.float32),
            pltpu.VMEM((GK * BW, DIM), jnp.float32),
            pltpu.VMEM((GK * BW, DIM), jnp.float32),
            pltpu.SemaphoreType.DMA,
            pltpu.SemaphoreType.DMA,
            pltpu.SemaphoreType.DMA,
            pltpu.SemaphoreType.DMA,
        ],
    )(pair_u, pair_v, pos_u, neg_u, mask_pos, mask_neg, u_table, v_table)


def _fin_body(dots_ref, out_ref):
    x = dots_ref[...]
    row = lax.broadcasted_iota(jnp.int32, (K, B), 0)
    s = jnp.where(row >= 1 + KN, -x, x)
    t = jnp.minimum(s, 0.0) - jnp.log1p(jnp.exp(-jnp.abs(s)))
    coef = jnp.where(row == 0, jnp.float32(1 + KN),
                     jnp.where(row >= 1 + KN, jnp.float32(1.0),
                               jnp.float32(-1.0)))
    out_ref[0, 0] = -jnp.sum(coef * t)


def _finish(dots):
    return pl.pallas_call(
        _fin_body,
        out_shape=jax.ShapeDtypeStruct((1, 1), jnp.float32),
        in_specs=[pl.BlockSpec(memory_space=pltpu.VMEM)],
        out_specs=pl.BlockSpec(memory_space=pltpu.SMEM),
    )(dots)


def kernel(pair_u, pair_v, pos_u, mask_pos_u, neg_u, mask_neg_u,
           u_table, v_table):
    dots = _sc_dots(pair_u.astype(jnp.int32), pair_v.astype(jnp.int32),
                    pos_u.astype(jnp.int32), neg_u.astype(jnp.int32),
                    mask_pos_u, mask_neg_u, u_table, v_table)
    return _finish(dots)[0, 0]
